# baseline (speedup 1.0000x reference)
import functools

import jax
import jax.numpy as jnp
from jax import lax
from jax.experimental import pallas as pl
from jax.experimental.pallas import tpu as pltpu
from jax.experimental.pallas import tpu_sc as plsc

INST = 32
THR = 0.01


_S = 72
_PW = 66 * _S + 2


def _conv_body(x, w1, w2, b1, b2, o, xp):
    xb = x[0].astype(jnp.bfloat16)
    xp[...] = jnp.zeros((256, _PW), jnp.bfloat16)
    for y in range(64):
        xp[:, pl.ds((y + 1) * _S + 1, 64)] = xb[:, y * 64:(y + 1) * 64]
    acc = None
    for dy in range(3):
        for dx in range(3):
            w = w1[pl.ds((dy * 3 + dx) * 512, 512), :]
            xs = xp[:, pl.ds(dy * _S + dx, 64 * _S)]
            t = lax.dot_general(w, xs, (((1,), (0,)), ((), ())),
                                preferred_element_type=jnp.float32)
            acc = t if acc is None else acc + t
    r = jnp.maximum(acc + b1[...], 0.0).astype(jnp.bfloat16)
    logits = lax.dot_general(w2[...], r, (((1,), (0,)), ((), ())),
                             preferred_element_type=jnp.float32) + b2[...]
    o[0] = jax.nn.sigmoid(logits)


def _conv_heads(x, w1, w2, b1, b2, B):
    return pl.pallas_call(
        _conv_body,
        grid=(B,),
        in_specs=[
            pl.BlockSpec((1, 256, 4096), lambda b: (b, 0, 0)),
            pl.BlockSpec((9 * 512, 256), lambda b: (0, 0)),
            pl.BlockSpec((8, 512), lambda b: (0, 0)),
            pl.BlockSpec((512, 1), lambda b: (0, 0)),
            pl.BlockSpec((8, 1), lambda b: (0, 0)),
        ],
        out_specs=pl.BlockSpec((1, 8, 64 * _S), lambda b: (b, 0, 0)),
        out_shape=jax.ShapeDtypeStruct((B, 8, 64 * _S), jnp.float32),
        scratch_shapes=[pltpu.VMEM((256, _PW), jnp.bfloat16)],
    )(x, w1, w2, b1, b2)


def _sc_tail(out):
    B = out.shape[0] // 8
    W = 64
    S = 72
    CAP = 8 * W + 16
    MX = 64 * W + 16
    mesh = plsc.VectorSubcoreMesh(core_axis_name="c", subcore_axis_name="s")

    @functools.partial(
        pl.kernel,
        out_type=[jax.ShapeDtypeStruct((B, INST), jnp.float32),
                  jax.ShapeDtypeStruct((B, 2 * INST), jnp.float32),
                  jax.ShapeDtypeStruct((4 * 8, CAP), jnp.float32),
                  jax.ShapeDtypeStruct((4 * 8, CAP), jnp.int32),
                  jax.ShapeDtypeStruct((4 * 8, 16), jnp.int32)],
        mesh=mesh,
        compiler_params=pltpu.CompilerParams(needs_layout_passes=False),
        scratch_types=[
            pltpu.VMEM((4608,), jnp.float32),
            pltpu.VMEM((8 * W,), jnp.float32),
            pltpu.VMEM((CAP,), jnp.float32),
            pltpu.VMEM((CAP,), jnp.int32),
            pltpu.VMEM((16,), jnp.int32),
            pltpu.VMEM((CAP,), jnp.float32),
            pltpu.VMEM((CAP,), jnp.int32),
            pltpu.VMEM((MX,), jnp.float32),
            pltpu.VMEM((MX,), jnp.int32),
            pltpu.VMEM((4608,), jnp.float32),
            pltpu.VMEM((4608,), jnp.float32),
            pltpu.VMEM((16,), jnp.float32),
            pltpu.VMEM((16,), jnp.int32),
            pltpu.VMEM((16,), jnp.int32),
            pltpu.VMEM((INST,), jnp.float32),
            pltpu.VMEM((INST,), jnp.int32),
            pltpu.VMEM((INST,), jnp.float32),
            pltpu.VMEM((2 * INST,), jnp.float32),
        ],
    )
    def tail(out_h, conf_h, pp_h, shs, shp, shc,
             hm_v, vm_v, cs_v, cp_v, cnt_v, ms_v, mp_v, gs_v, gp_v,
             o0_v, o1_v, bfv_v, bfp_v, bfs_v, wv_v, wp_v, conf_v, pp_v):
        c = lax.axis_index("c")
        s = lax.axis_index("s")
        b = c * 2 + s // 8
        t = s % 8
        li = s // 8
        lanes = lax.iota(jnp.int32, 16)

        y0 = t * 8
        pltpu.sync_copy(out_h.at[b * 8], hm_v)

        for r in range(8):
            y = y0 + r
            up = jnp.where(y == 0, y, y - 1)
            dn = jnp.where(y == 63, y, y + 1)
            for j in range(4):
                o = j * 16
                v = jnp.maximum(hm_v[pl.ds(up * S + o, 16)],
                                hm_v[pl.ds(y * S + o, 16)])
                vm_v[pl.ds(r * W + o, 16)] = jnp.maximum(
                    v, hm_v[pl.ds(dn * S + o, 16)])

        nl = 0
        for r in range(8):
            lr = y0 + r
            row = r * W
            for j in range(4):
                o = j * 16
                sv = hm_v[pl.ds(lr * S + o, 16)]
                cv = vm_v[pl.ds(row + o, 16)]
                xidx = row + o + lanes
                left = plsc.load_gather(vm_v, [jnp.maximum(xidx - 1, row)])
                right = plsc.load_gather(
                    vm_v, [jnp.minimum(xidx + 1, row + W - 1)])
                hmax = jnp.maximum(jnp.maximum(left, cv), right)
                pk = (sv == hmax) & (sv > THR)
                pix = (y0 + r) * W + o + lanes
                plsc.store_compressed(cs_v.at[pl.ds(nl, 16)], sv, mask=pk)
                plsc.store_compressed(cp_v.at[pl.ds(nl, 16)], pix, mask=pk)
                nl = nl + plsc.all_reduce_population_count(pk)[0]

        cnt_v[...] = jnp.zeros((16,), jnp.int32) + nl
        gt = b * 8 + t
        pltpu.sync_copy(cs_v, shs.at[gt])
        pltpu.sync_copy(cp_v, shp.at[gt])
        pltpu.sync_copy(cnt_v, shc.at[gt])
        plsc.subcore_barrier()

        @pl.when(t == 0)
        def _():
            pltpu.sync_copy(out_h.at[b * 8 + 1], o0_v)
            pltpu.sync_copy(out_h.at[b * 8 + 2], o1_v)

            n = 0
            for t2 in range(8):
                pltpu.sync_copy(shc.at[b * 8 + t2], cnt_v)
                ct = jnp.minimum(jnp.maximum(cnt_v[...][0], 0), 8 * W)
                pltpu.sync_copy(shs.at[b * 8 + t2], ms_v)
                pltpu.sync_copy(shp.at[b * 8 + t2], mp_v)

                def mblk(bi, nn):
                    rem = ct - bi * 16
                    msk = lanes < rem
                    plsc.store_compressed(gs_v.at[pl.ds(nn, 16)],
                                          ms_v[pl.ds(bi * 16, 16)], mask=msk)
                    plsc.store_compressed(gp_v.at[pl.ds(nn, 16)],
                                          mp_v[pl.ds(bi * 16, 16)], mask=msk)
                    return nn + jnp.minimum(rem, 16)
                n = lax.fori_loop(0, (ct + 15) // 16, mblk, n)

            gs_v[pl.ds(n, 16)] = jnp.zeros((16,), jnp.float32)

            nb = (n + 15) // 16
            big = jnp.full((16,), 1 << 30, jnp.int32)

            def pick(kk, carry):
                def scan_block(bi, st):
                    bv, bp, bs = st
                    v = gs_v[pl.ds(bi * 16, 16)]
                    pix = gp_v[pl.ds(bi * 16, 16)]
                    slot = bi * 16 + lanes
                    better = (v > bv) | ((v == bv) & (pix < bp))
                    return (jnp.where(better, v, bv),
                            jnp.where(better, pix, bp),
                            jnp.where(better, slot, bs))
                bv, bp, bs = lax.fori_loop(
                    0, nb, scan_block,
                    (jnp.zeros((16,), jnp.float32), big,
                     jnp.zeros((16,), jnp.int32)))
                for sh in (8, 4, 2, 1):
                    bfv_v[...] = bv
                    bfp_v[...] = bp
                    bfs_v[...] = bs
                    pidx = lanes ^ sh
                    ov = plsc.load_gather(bfv_v, [pidx])
                    op = plsc.load_gather(bfp_v, [pidx])
                    osl = plsc.load_gather(bfs_v, [pidx])
                    better = (ov > bv) | ((ov == bv) & (op < bp))
                    bv = jnp.where(better, ov, bv)
                    bp = jnp.where(better, op, bp)
                    bs = jnp.where(better, osl, bs)
                lane0 = lanes == 0
                kvec = jnp.zeros((16,), jnp.int32) + kk
                plsc.store_scatter(wv_v, [kvec], bv, mask=lane0)
                plsc.store_scatter(wp_v, [kvec], bp, mask=lane0)
                plsc.store_scatter(gs_v, [bs],
                                   jnp.zeros((16,), jnp.float32), mask=lane0)
                return carry
            lax.fori_loop(0, INST, pick, 0)

            for h in range(2):
                wv = wv_v[pl.ds(h * 16, 16)]
                wp = wp_v[pl.ds(h * 16, 16)]
                valid = wv > 0.0
                pix = jnp.where(valid, wp, 0)
                ysi = pix // W
                q = pix + ysi * (S - W)
                ysf = ysi.astype(jnp.float32)
                xsf = (pix - ysi * W).astype(jnp.float32)
                g0 = plsc.load_gather(o0_v, [q])
                g1 = plsc.load_gather(o1_v, [q])
                conf_v[pl.ds(h * 16, 16)] = wv
                ei = (h * 16 + lanes) * 2
                plsc.store_scatter(pp_v, [ei], (ysf + g0) / 63.0)
                plsc.store_scatter(pp_v, [ei + 1], (xsf + g1) / 63.0)

            pltpu.sync_copy(conf_v, conf_h.at[b])
            pltpu.sync_copy(pp_v, pp_h.at[b])

    return tail(out)[:2]


def kernel(features, W1h, b1h, W2h, b2h, W1o, b1o, W2o, b2o):
    B, C, H, W = features.shape
    HEAD = W1h.shape[0]
    x = features.reshape(B, C, H * W)
    w1 = jnp.concatenate([W1h, W1o], axis=0).transpose(2, 3, 0, 1) \
        .reshape(9 * 2 * HEAD, C).astype(jnp.bfloat16)
    w2 = jnp.zeros((8, 2 * HEAD), jnp.float32)
    w2 = w2.at[0, :HEAD].set(W2h.reshape(HEAD))
    w2 = w2.at[1, HEAD:].set(W2o.reshape(2, HEAD)[0])
    w2 = w2.at[2, HEAD:].set(W2o.reshape(2, HEAD)[1]).astype(jnp.bfloat16)
    b1 = jnp.concatenate([b1h, b1o]).reshape(2 * HEAD, 1)
    b2 = jnp.zeros((8, 1), jnp.float32).at[0, 0].set(b2h[0]) \
        .at[1, 0].set(b2o[0]).at[2, 0].set(b2o[1])

    out = _conv_heads(x, w1, w2, b1, b2, B)
    outs = out.reshape(B, 8, H, _S)[:, :, :, :W]

    pred_hm = outs[:, 0:1]
    pred_offset = outs[:, 1:3]

    conf, pp = _sc_tail(out.reshape(B * 8, 64 * _S))
    return pred_hm, pred_offset, conf, pp.reshape(B, INST, 2)


# speedup vs baseline: 1.0865x; 1.0865x over previous
import functools

import jax
import jax.numpy as jnp
from jax import lax
from jax.experimental import pallas as pl
from jax.experimental.pallas import tpu as pltpu
from jax.experimental.pallas import tpu_sc as plsc

INST = 32
THR = 0.01


_S = 72
_PW = 66 * _S + 2


def _conv_body(x, w1, w2, b1, b2, o, xp):
    xb = x[0].astype(jnp.bfloat16)
    xp[...] = jnp.zeros((256, _PW), jnp.bfloat16)
    for y in range(64):
        xp[:, pl.ds((y + 1) * _S + 1, 64)] = xb[:, y * 64:(y + 1) * 64]
    acc = None
    for dy in range(3):
        for dx in range(3):
            w = w1[pl.ds((dy * 3 + dx) * 512, 512), :]
            xs = xp[:, pl.ds(dy * _S + dx, 64 * _S)]
            t = lax.dot_general(w, xs, (((1,), (0,)), ((), ())),
                                preferred_element_type=jnp.float32)
            acc = t if acc is None else acc + t
    r = jnp.maximum(acc + b1[...], 0.0).astype(jnp.bfloat16)
    logits = lax.dot_general(w2[...], r, (((1,), (0,)), ((), ())),
                             preferred_element_type=jnp.float32) + b2[...]
    o[0] = jax.nn.sigmoid(logits)


def _conv_heads(x, w1, w2, b1, b2, B):
    return pl.pallas_call(
        _conv_body,
        grid=(B,),
        in_specs=[
            pl.BlockSpec((1, 256, 4096), lambda b: (b, 0, 0)),
            pl.BlockSpec((9 * 512, 256), lambda b: (0, 0)),
            pl.BlockSpec((8, 512), lambda b: (0, 0)),
            pl.BlockSpec((512, 1), lambda b: (0, 0)),
            pl.BlockSpec((8, 1), lambda b: (0, 0)),
        ],
        out_specs=pl.BlockSpec((1, 8, 64 * _S), lambda b: (b, 0, 0)),
        out_shape=jax.ShapeDtypeStruct((B, 8, 64 * _S), jnp.float32),
        scratch_shapes=[pltpu.VMEM((256, _PW), jnp.bfloat16)],
    )(x, w1, w2, b1, b2)


def _sc_tail(out):
    B = out.shape[0] // 8
    W = 64
    S = 72
    CAP = 8 * W + 16
    MX = 64 * W + 16
    mesh = plsc.VectorSubcoreMesh(core_axis_name="c", subcore_axis_name="s")

    @functools.partial(
        pl.kernel,
        out_type=[jax.ShapeDtypeStruct((B, INST), jnp.float32),
                  jax.ShapeDtypeStruct((B, 2 * INST), jnp.float32),
                  jax.ShapeDtypeStruct((4 * 8, CAP), jnp.float32),
                  jax.ShapeDtypeStruct((4 * 8, CAP), jnp.int32),
                  jax.ShapeDtypeStruct((4 * 8, 16), jnp.int32)],
        mesh=mesh,
        compiler_params=pltpu.CompilerParams(needs_layout_passes=False),
        scratch_types=[
            pltpu.VMEM((4608,), jnp.float32),
            pltpu.VMEM((8 * W,), jnp.float32),
            pltpu.VMEM((CAP,), jnp.float32),
            pltpu.VMEM((CAP,), jnp.int32),
            pltpu.VMEM((16,), jnp.int32),
            pltpu.VMEM((8, CAP), jnp.float32),
            pltpu.VMEM((8, CAP), jnp.int32),
            pltpu.VMEM((8, 16), jnp.int32),
            pltpu.VMEM((MX,), jnp.float32),
            pltpu.VMEM((MX,), jnp.int32),
            pltpu.VMEM((4608,), jnp.float32),
            pltpu.VMEM((4608,), jnp.float32),
            pltpu.VMEM((16,), jnp.float32),
            pltpu.VMEM((16,), jnp.int32),
            pltpu.VMEM((16,), jnp.int32),
            pltpu.VMEM((INST,), jnp.float32),
            pltpu.VMEM((INST,), jnp.int32),
            pltpu.VMEM((INST,), jnp.float32),
            pltpu.VMEM((2 * INST,), jnp.float32),
        ],
    )
    def tail(out_h, conf_h, pp_h, shs, shp, shc,
             hm_v, vm_v, cs_v, cp_v, cnt_v, ms_v, mp_v, cnts_v, gs_v, gp_v,
             o0_v, o1_v, bfv_v, bfp_v, bfs_v, wv_v, wp_v, conf_v, pp_v):
        c = lax.axis_index("c")
        s = lax.axis_index("s")
        b = c * 2 + s // 8
        t = s % 8
        li = s // 8
        lanes = lax.iota(jnp.int32, 16)

        y0 = t * 8
        pltpu.sync_copy(out_h.at[b * 8], hm_v)

        for r in range(8):
            y = y0 + r
            up = jnp.where(y == 0, y, y - 1)
            dn = jnp.where(y == 63, y, y + 1)
            for j in range(4):
                o = j * 16
                v = jnp.maximum(hm_v[pl.ds(up * S + o, 16)],
                                hm_v[pl.ds(y * S + o, 16)])
                vm_v[pl.ds(r * W + o, 16)] = jnp.maximum(
                    v, hm_v[pl.ds(dn * S + o, 16)])

        nl = 0
        for r in range(8):
            lr = y0 + r
            row = r * W
            for j in range(4):
                o = j * 16
                sv = hm_v[pl.ds(lr * S + o, 16)]
                cv = vm_v[pl.ds(row + o, 16)]
                xidx = row + o + lanes
                left = plsc.load_gather(vm_v, [jnp.maximum(xidx - 1, row)])
                right = plsc.load_gather(
                    vm_v, [jnp.minimum(xidx + 1, row + W - 1)])
                hmax = jnp.maximum(jnp.maximum(left, cv), right)
                pk = (sv == hmax) & (sv > THR)
                pix = (y0 + r) * W + o + lanes
                plsc.store_compressed(cs_v.at[pl.ds(nl, 16)], sv, mask=pk)
                plsc.store_compressed(cp_v.at[pl.ds(nl, 16)], pix, mask=pk)
                nl = nl + plsc.all_reduce_population_count(pk)[0]

        cnt_v[...] = jnp.zeros((16,), jnp.int32) + nl
        gt = b * 8 + t
        pltpu.sync_copy(cs_v, shs.at[gt])
        pltpu.sync_copy(cp_v, shp.at[gt])
        pltpu.sync_copy(cnt_v, shc.at[gt])
        plsc.subcore_barrier()

        @pl.when(t == 0)
        def _():
            pltpu.sync_copy(out_h.at[b * 8 + 1], o0_v)
            pltpu.sync_copy(out_h.at[b * 8 + 2], o1_v)

            pltpu.sync_copy(shc.at[pl.ds(b * 8, 8)], cnts_v)
            pltpu.sync_copy(shs.at[pl.ds(b * 8, 8)], ms_v)
            pltpu.sync_copy(shp.at[pl.ds(b * 8, 8)], mp_v)
            n = 0
            for t2 in range(8):
                ct = cnts_v[t2, pl.ds(0, 16)][0]

                def mblk(bi, nn):
                    rem = ct - bi * 16
                    msk = lanes < rem
                    plsc.store_compressed(gs_v.at[pl.ds(nn, 16)],
                                          ms_v[t2, pl.ds(bi * 16, 16)],
                                          mask=msk)
                    plsc.store_compressed(gp_v.at[pl.ds(nn, 16)],
                                          mp_v[t2, pl.ds(bi * 16, 16)],
                                          mask=msk)
                    return nn + jnp.minimum(rem, 16)
                n = lax.fori_loop(0, (ct + 15) // 16, mblk, n)

            gs_v[pl.ds(n, 16)] = jnp.zeros((16,), jnp.float32)

            nb = (n + 15) // 16
            big = jnp.full((16,), 1 << 30, jnp.int32)

            def pick(kk, carry):
                def scan_block(bi, st):
                    bv, bp, bs = st
                    v = gs_v[pl.ds(bi * 16, 16)]
                    pix = gp_v[pl.ds(bi * 16, 16)]
                    slot = bi * 16 + lanes
                    better = (v > bv) | ((v == bv) & (pix < bp))
                    return (jnp.where(better, v, bv),
                            jnp.where(better, pix, bp),
                            jnp.where(better, slot, bs))
                bv, bp, bs = lax.fori_loop(
                    0, nb, scan_block,
                    (jnp.zeros((16,), jnp.float32), big,
                     jnp.zeros((16,), jnp.int32)))
                for sh in (8, 4, 2, 1):
                    bfv_v[...] = bv
                    bfp_v[...] = bp
                    bfs_v[...] = bs
                    pidx = lanes ^ sh
                    ov = plsc.load_gather(bfv_v, [pidx])
                    op = plsc.load_gather(bfp_v, [pidx])
                    osl = plsc.load_gather(bfs_v, [pidx])
                    better = (ov > bv) | ((ov == bv) & (op < bp))
                    bv = jnp.where(better, ov, bv)
                    bp = jnp.where(better, op, bp)
                    bs = jnp.where(better, osl, bs)
                lane0 = lanes == 0
                kvec = jnp.zeros((16,), jnp.int32) + kk
                plsc.store_scatter(wv_v, [kvec], bv, mask=lane0)
                plsc.store_scatter(wp_v, [kvec], bp, mask=lane0)
                plsc.store_scatter(gs_v, [bs],
                                   jnp.zeros((16,), jnp.float32), mask=lane0)
                return carry
            lax.fori_loop(0, INST, pick, 0)

            for h in range(2):
                wv = wv_v[pl.ds(h * 16, 16)]
                wp = wp_v[pl.ds(h * 16, 16)]
                valid = wv > 0.0
                pix = jnp.where(valid, wp, 0)
                ysi = pix // W
                q = pix + ysi * (S - W)
                ysf = ysi.astype(jnp.float32)
                xsf = (pix - ysi * W).astype(jnp.float32)
                g0 = plsc.load_gather(o0_v, [q])
                g1 = plsc.load_gather(o1_v, [q])
                conf_v[pl.ds(h * 16, 16)] = wv
                ei = (h * 16 + lanes) * 2
                plsc.store_scatter(pp_v, [ei], (ysf + g0) / 63.0)
                plsc.store_scatter(pp_v, [ei + 1], (xsf + g1) / 63.0)

            pltpu.sync_copy(conf_v, conf_h.at[b])
            pltpu.sync_copy(pp_v, pp_h.at[b])

    return tail(out)[:2]


def kernel(features, W1h, b1h, W2h, b2h, W1o, b1o, W2o, b2o):
    B, C, H, W = features.shape
    HEAD = W1h.shape[0]
    x = features.reshape(B, C, H * W)
    w1 = jnp.concatenate([W1h, W1o], axis=0).transpose(2, 3, 0, 1) \
        .reshape(9 * 2 * HEAD, C).astype(jnp.bfloat16)
    w2 = jnp.zeros((8, 2 * HEAD), jnp.float32)
    w2 = w2.at[0, :HEAD].set(W2h.reshape(HEAD))
    w2 = w2.at[1, HEAD:].set(W2o.reshape(2, HEAD)[0])
    w2 = w2.at[2, HEAD:].set(W2o.reshape(2, HEAD)[1]).astype(jnp.bfloat16)
    b1 = jnp.concatenate([b1h, b1o]).reshape(2 * HEAD, 1)
    b2 = jnp.zeros((8, 1), jnp.float32).at[0, 0].set(b2h[0]) \
        .at[1, 0].set(b2o[0]).at[2, 0].set(b2o[1])

    out = _conv_heads(x, w1, w2, b1, b2, B)
    outs = out.reshape(B, 8, H, _S)[:, :, :, :W]

    pred_hm = outs[:, 0:1]
    pred_offset = outs[:, 1:3]

    conf, pp = _sc_tail(out.reshape(B * 8, 64 * _S))
    return pred_hm, pred_offset, conf, pp.reshape(B, INST, 2)
